# baseline (device time: 75281 ns/iter reference)
import jax
import jax.numpy as jnp
from jax import lax
from jax.experimental import pallas as pl
from jax.experimental.pallas import tpu as pltpu

N_DEV = 8


def kernel(x, W1, W2):
    m_per, d = x.shape
    _, f_per = W1.shape

    def body(x_ref, w1_ref, w2_ref, out_ref,
             xg_ref, cbuf_ref, rs_ref,
             ag_send_sem, ag_recv_sem, rs_send_sem, rs_recv_sem):
        i = lax.axis_index("i")

        barrier = pltpu.get_barrier_semaphore()
        for o in range(1, N_DEV):
            peer = lax.rem(i + o, N_DEV)
            pl.semaphore_signal(barrier, inc=1, device_id=(peer,),
                                device_id_type=pl.DeviceIdType.MESH)
        pl.semaphore_wait(barrier, N_DEV - 1)

        w1b = w1_ref[...].astype(jnp.bfloat16)
        w2b = w2_ref[...].astype(jnp.bfloat16)
        xg_ref[N_DEV - 1] = x_ref[...].astype(jnp.bfloat16)

        def contrib(xb):
            h = jnp.dot(xb, w1b, preferred_element_type=jnp.float32)
            h = h * jax.nn.sigmoid(h)
            return jnp.dot(h.astype(jnp.bfloat16), w2b,
                           preferred_element_type=jnp.float32)

        pending = []
        for o in range(1, N_DEV):
            peer = lax.rem(i + o, N_DEV)
            rdma = pltpu.make_async_remote_copy(
                src_ref=xg_ref.at[N_DEV - 1],
                dst_ref=xg_ref.at[N_DEV - 1 - o],
                send_sem=ag_send_sem.at[o - 1],
                recv_sem=ag_recv_sem.at[N_DEV - 1 - o],
                device_id=(peer,),
                device_id_type=pl.DeviceIdType.MESH,
            )
            rdma.start()
            pending.append(rdma)

        c_own = contrib(xg_ref[N_DEV - 1])

        for k in range(N_DEV - 1):
            recv = pltpu.make_async_remote_copy(
                src_ref=xg_ref.at[N_DEV - 1],
                dst_ref=xg_ref.at[k],
                send_sem=ag_send_sem.at[0],
                recv_sem=ag_recv_sem.at[k],
                device_id=(i,),
                device_id_type=pl.DeviceIdType.MESH,
            )
            recv.wait_recv()
            cbuf_ref[k] = contrib(xg_ref[k]).astype(jnp.bfloat16)
            owner = lax.rem(i + k + 1, N_DEV)
            rdma = pltpu.make_async_remote_copy(
                src_ref=cbuf_ref.at[k],
                dst_ref=rs_ref.at[N_DEV - 2 - k],
                send_sem=rs_send_sem.at[k],
                recv_sem=rs_recv_sem.at[N_DEV - 2 - k],
                device_id=(owner,),
                device_id_type=pl.DeviceIdType.MESH,
            )
            rdma.start()
            pending.append(rdma)

        acc = c_own
        for m in range(N_DEV - 1):
            recv = pltpu.make_async_remote_copy(
                src_ref=cbuf_ref.at[0],
                dst_ref=rs_ref.at[m],
                send_sem=rs_send_sem.at[0],
                recv_sem=rs_recv_sem.at[m],
                device_id=(i,),
                device_id_type=pl.DeviceIdType.MESH,
            )
            recv.wait_recv()
            acc = acc + rs_ref[m].astype(jnp.float32)
        out_ref[...] = acc

        for rdma in pending:
            rdma.wait_send()

    return pl.pallas_call(
        body,
        out_shape=jax.ShapeDtypeStruct((m_per, d), jnp.float32),
        in_specs=[
            pl.BlockSpec(memory_space=pltpu.VMEM),
            pl.BlockSpec(memory_space=pltpu.VMEM),
            pl.BlockSpec(memory_space=pltpu.VMEM),
        ],
        out_specs=pl.BlockSpec(memory_space=pltpu.VMEM),
        scratch_shapes=[
            pltpu.VMEM((N_DEV, m_per, d), jnp.bfloat16),
            pltpu.VMEM((N_DEV - 1, m_per, d), jnp.bfloat16),
            pltpu.VMEM((N_DEV - 1, m_per, d), jnp.bfloat16),
            pltpu.SemaphoreType.DMA((N_DEV - 1,)),
            pltpu.SemaphoreType.DMA((N_DEV - 1,)),
            pltpu.SemaphoreType.DMA((N_DEV - 1,)),
            pltpu.SemaphoreType.DMA((N_DEV - 1,)),
        ],
        compiler_params=pltpu.CompilerParams(collective_id=0),
    )(x, W1, W2)
